# baseline (speedup 1.0000x reference)
import functools

import jax
import jax.numpy as jnp
from jax import lax
from jax.experimental import pallas as pl
from jax.experimental.pallas import tpu as pltpu
from jax.experimental.pallas import tpu_sc as plsc

_LANES = 128
_NBUF = 4
_TROWS = 8064
_G16 = 16


@functools.partial(jax.jit, static_argnums=(2,))
def _gather_rows(x_flat2d, pe, steps_per_worker):
    D = pe.shape[1]
    B_total = x_flat2d.size
    n_steps = steps_per_worker
    n_hi = pe.shape[0] - _TROWS
    mesh = plsc.VectorSubcoreMesh(core_axis_name="c", subcore_axis_name="s")

    @functools.partial(
        pl.kernel,
        mesh=mesh,
        out_type=jax.ShapeDtypeStruct((B_total, D), jnp.float32),
        scratch_types=[
            pltpu.VMEM((_NBUF, _LANES), jnp.int32),
            pltpu.VMEM((_NBUF, _LANES), jnp.int32),
            pltpu.VMEM((_NBUF * _LANES, D), jnp.float32),
            pltpu.VMEM_SHARED((_TROWS, D), jnp.float32),
        ] + [pltpu.SemaphoreType.DMA] * (3 * _NBUF + 1),
    )
    def k(x_hbm, pe_hbm, out_hbm, idx_b, cl_b, rows_v, pe_sp, *sems):
        sem_i = sems[:_NBUF]
        sem_g = sems[_NBUF:2 * _NBUF]
        sem_w = sems[2 * _NBUF:3 * _NBUF]
        sem_f = sems[3 * _NBUF]
        n_cores = lax.axis_size("c")
        n_sub = lax.axis_size("s")
        sid = lax.axis_index("s")
        wid = sid * n_cores + lax.axis_index("c")
        row_base = wid * n_steps

        stripe = _TROWS // n_sub
        pltpu.sync_copy(
            pe_hbm.at[pl.ds(sid * stripe, stripe)],
            pe_sp.at[pl.ds(sid * stripe, stripe)],
        )
        plsc.subcore_barrier()

        def buf(b):
            return rows_v.at[pl.ds(b * _LANES, _LANES)]

        def idx_src(s):
            return x_hbm.at[pl.ds(row_base + s, 1)]

        def fire_idx(s, b):
            pltpu.async_copy(idx_src(s), idx_b.at[pl.ds(b, 1)], sem_i[b])

        def wait_idx(s, b):
            pltpu.make_async_copy(
                idx_src(s), idx_b.at[pl.ds(b, 1)], sem_i[b]
            ).wait()

        def clamp_idx(b):
            for g in range(_LANES // _G16):
                grp = idx_b[b, pl.ds(g * _G16, _G16)]
                cl_b[b, pl.ds(g * _G16, _G16)] = jnp.minimum(grp, _TROWS - 1)

        def fire_gather(s, b):
            pltpu.async_copy(pe_sp.at[cl_b.at[b]], buf(b), sem_g[b])

        def wait_gather(s, b):
            pltpu.make_async_copy(
                pe_sp.at[cl_b.at[b]], buf(b), sem_g[b]
            ).wait()

        def fire_fixups(b):
            def grp_body(g, cnt):
                grp = idx_b[b, pl.ds(g * _G16, _G16)]
                n_g = jnp.int32(0)
                for l in range(_G16):
                    n_g = n_g + jnp.where(
                        grp[l] >= _TROWS, jnp.int32(1), jnp.int32(0)
                    )

                @pl.when(n_g > 0)
                def _():
                    for l in range(_G16):
                        v_l = grp[l]

                        @pl.when(v_l >= _TROWS)
                        def _():
                            row = b * _LANES + g * _G16 + l
                            pltpu.async_copy(
                                pe_hbm.at[pl.ds(v_l, 1)],
                                rows_v.at[pl.ds(row, 1)],
                                sem_f,
                            )

                return cnt + n_g

            return lax.fori_loop(0, _LANES // _G16, grp_body, jnp.int32(0))

        def wait_fixups(cnt):
            def wbody(i, c):
                pltpu.make_async_copy(
                    pe_hbm.at[pl.ds(0, 1)], rows_v.at[pl.ds(0, 1)], sem_f
                ).wait()
                return c

            lax.fori_loop(0, cnt, wbody, 0)

        def out_dst(s):
            return out_hbm.at[pl.ds((row_base + s) * _LANES, _LANES)]

        def fire_write(s, b):
            pltpu.async_copy(buf(b), out_dst(s), sem_w[b])

        def wait_write(s, b):
            pltpu.make_async_copy(buf(b), out_dst(s), sem_w[b]).wait()

        for b in range(_NBUF):
            fire_idx(b, b)
        for b in range(3):
            wait_idx(b, b)
            clamp_idx(b)
            fire_gather(b, b)

        def body(g, carry):
            for b in range(_NBUF):
                s = 4 * g + b
                wait_gather(s, b)
                n_fix = fire_fixups(b)

                s4 = s + 4

                @pl.when(s4 < n_steps)
                def _():
                    fire_idx(s4, b)

                b2 = (b + 3) % _NBUF
                s3 = s + 3

                @pl.when(s3 < n_steps)
                def _():
                    @pl.when(s3 >= _NBUF)
                    def _():
                        wait_write(s3 - _NBUF, b2)
                    wait_idx(s3, b2)
                    clamp_idx(b2)
                    fire_gather(s3, b2)

                wait_fixups(n_fix)
                fire_write(s, b)

            return carry

        lax.fori_loop(0, n_steps // _NBUF, body, 0)

        for b in range(_NBUF):
            wait_write(n_steps - _NBUF + b, b)

    return k(x_flat2d, pe)


def kernel(x, pe):
    B, H = x.shape
    D = pe.shape[1]
    total = B * H
    info = plsc.get_sparse_core_info()
    n_workers = info.num_cores * info.num_subcores
    assert total % (n_workers * _LANES * _NBUF) == 0
    steps_per_worker = total // (n_workers * _LANES)
    x2 = jnp.reshape(x.astype(jnp.int32), (total // _LANES, _LANES))
    out = _gather_rows(x2, pe, steps_per_worker)
    return jnp.reshape(out, (B, H, D))
